# baseline (speedup 1.0000x reference)
import functools
import math

import jax
import jax.numpy as jnp
import numpy as np
from jax import lax
from jax.experimental import pallas as pl
from jax.experimental.pallas import tpu as pltpu
from jax.experimental.pallas import tpu_sc as plsc

VOCAB = 100000
D_MODEL = 1024
BATCH = 4
SEQ = 4096
NTOK = BATCH * SEQ

NUM_CORES = 2
NUM_SUBCORES = 16
NW = NUM_CORES * NUM_SUBCORES
TPW = NTOK // NW
CHUNK = 8
NCHUNK = TPW // CHUNK
NBUF = 4
LANES = 16
VPR = D_MODEL // LANES


def _make_pe(seq_len: int, d_model: int) -> np.ndarray:
    pe = np.zeros((seq_len, d_model), dtype=np.float32)
    position = np.arange(0, seq_len, dtype=np.float32)[:, None]
    div_term = np.exp(
        np.arange(0, d_model, 2, dtype=np.float32) * -(math.log(10000.0) / d_model)
    )
    pe[:, 0::2] = np.sin(position * div_term)
    pe[:, 1::2] = np.cos(position * div_term)
    return pe


_PE = _make_pe(SEQ, D_MODEL)
_SCALE = math.sqrt(D_MODEL)


def _emb_body(emb_hbm, idx_hbm, pe_hbm, out_hbm, idx_v,
              r0, r1, r2, r3, p0, p1, p2, p3,
              gs0, gs1, gs2, gs3, ps0, ps1, ps2, ps3,
              os0, os1, os2, os3):
    rows = (r0, r1, r2, r3)
    pebs = (p0, p1, p2, p3)
    gsems = (gs0, gs1, gs2, gs3)
    psems = (ps0, ps1, ps2, ps3)
    osems = (os0, os1, os2, os3)

    wid = lax.axis_index("s") * NUM_CORES + lax.axis_index("c")
    tbase = wid * TPW
    pbase = lax.rem(tbase, SEQ)

    pltpu.sync_copy(idx_hbm.at[pl.ds(tbase, TPW)], idx_v)

    def fire(j, s):
        pltpu.async_copy(pe_hbm.at[pl.ds(pbase + j * CHUNK, CHUNK)],
                         pebs[s], psems[s])
        pltpu.async_copy(emb_hbm.at[idx_v.at[pl.ds(j * CHUNK, CHUNK)]],
                         rows[s], gsems[s])

    fire(0, 0)
    fire(1, 1)

    def group_step(g, _):
        for b in range(NBUF):
            j = g * NBUF + b
            j2 = j + 2
            s2 = (b + 2) % NBUF

            @pl.when(j2 < NCHUNK)
            def _():
                @pl.when(j2 >= NBUF)
                def _():
                    pltpu.make_async_copy(
                        pebs[s2], out_hbm.at[pl.ds(tbase, CHUNK)],
                        osems[s2]).wait()
                fire(j2, s2)

            pltpu.make_async_copy(
                emb_hbm.at[idx_v.at[pl.ds(j * CHUNK, CHUNK)]],
                rows[b], gsems[b]).wait()
            pltpu.make_async_copy(
                pe_hbm.at[pl.ds(pbase, CHUNK)], pebs[b], psems[b]).wait()

            rbuf = rows[b]
            pbuf = pebs[b]

            def fold(i, _):
                r = lax.shift_right_logical(i, 6)
                k = pl.multiple_of(
                    lax.shift_left(lax.bitwise_and(i, VPR - 1), 4), LANES)
                v = rbuf[r, pl.ds(k, LANES)]
                plsc.addupdate(pbuf.at[r, pl.ds(k, LANES)], v * _SCALE)
                return 0

            lax.fori_loop(0, CHUNK * VPR, fold, 0, unroll=16)

            pltpu.async_copy(pbuf, out_hbm.at[pl.ds(tbase + j * CHUNK, CHUNK)],
                             osems[b])
        return 0

    lax.fori_loop(0, NCHUNK // NBUF, group_step, 0)

    for b in range(NBUF):
        pltpu.make_async_copy(pebs[b], out_hbm.at[pl.ds(tbase, CHUNK)],
                              osems[b]).wait()


@jax.jit
def _emb_lookup(emb, idx, pe):
    mesh = plsc.VectorSubcoreMesh(core_axis_name="c", subcore_axis_name="s")
    kfn = pl.kernel(
        _emb_body,
        mesh=mesh,
        out_type=jax.ShapeDtypeStruct((NTOK, D_MODEL), jnp.float32),
        scratch_types=(
            [pltpu.VMEM((TPW,), jnp.int32)]
            + [pltpu.VMEM((CHUNK, D_MODEL), jnp.float32)] * (2 * NBUF)
            + [pltpu.SemaphoreType.DMA] * (3 * NBUF)
        ),
    )
    return kfn(emb, idx, pe)


def kernel(x, emb):
    idx = x.reshape(NTOK).astype(jnp.int32)
    pe = jnp.asarray(_PE)
    out = _emb_lookup(emb, idx, pe)
    return out.reshape(BATCH, SEQ, D_MODEL)


# speedup vs baseline: 1.0564x; 1.0564x over previous
import functools
import math

import jax
import jax.numpy as jnp
import numpy as np
from jax import lax
from jax.experimental import pallas as pl
from jax.experimental.pallas import tpu as pltpu
from jax.experimental.pallas import tpu_sc as plsc

VOCAB = 100000
D_MODEL = 1024
BATCH = 4
SEQ = 4096
NTOK = BATCH * SEQ

NUM_CORES = 2
NUM_SUBCORES = 16
NW = NUM_CORES * NUM_SUBCORES
TPW = NTOK // NW
CHUNK = 8
NCHUNK = TPW // CHUNK
NBUF = 4
LANES = 16
VPR = D_MODEL // LANES


def _make_pe(seq_len: int, d_model: int) -> np.ndarray:
    pe = np.zeros((seq_len, d_model), dtype=np.float32)
    position = np.arange(0, seq_len, dtype=np.float32)[:, None]
    div_term = np.exp(
        np.arange(0, d_model, 2, dtype=np.float32) * -(math.log(10000.0) / d_model)
    )
    pe[:, 0::2] = np.sin(position * div_term)
    pe[:, 1::2] = np.cos(position * div_term)
    return pe


_PE = _make_pe(SEQ, D_MODEL)
_SCALE = math.sqrt(D_MODEL)


def _emb_body(emb_hbm, idx_hbm, pe_hbm, out_hbm, idx_v,
              r0, r1, r2, r3, p0, p1, p2, p3,
              gs0, gs1, gs2, gs3, ps0, ps1, ps2, ps3,
              os0, os1, os2, os3):
    rows = (r0, r1, r2, r3)
    pebs = (p0, p1, p2, p3)
    gsems = (gs0, gs1, gs2, gs3)
    psems = (ps0, ps1, ps2, ps3)
    osems = (os0, os1, os2, os3)

    wid = lax.axis_index("s") * NUM_CORES + lax.axis_index("c")
    tbase = wid * TPW
    pbase = lax.rem(tbase, SEQ)

    pltpu.sync_copy(idx_hbm.at[pl.ds(tbase, TPW)], idx_v)

    def fire(j, s):
        pltpu.async_copy(pe_hbm.at[pl.ds(pbase + j * CHUNK, CHUNK)],
                         pebs[s], psems[s])
        pltpu.async_copy(emb_hbm.at[idx_v.at[pl.ds(j * CHUNK, CHUNK)]],
                         rows[s], gsems[s])

    fire(0, 0)
    fire(1, 1)

    def group_step(g, _):
        for b in range(NBUF):
            j = g * NBUF + b
            j2 = j + 2
            s2 = (b + 2) % NBUF

            @pl.when(j2 < NCHUNK)
            def _():
                @pl.when(j2 >= NBUF)
                def _():
                    pltpu.make_async_copy(
                        pebs[s2], out_hbm.at[pl.ds(tbase, CHUNK)],
                        osems[s2]).wait()
                fire(j2, s2)

            pltpu.make_async_copy(
                emb_hbm.at[idx_v.at[pl.ds(j * CHUNK, CHUNK)]],
                rows[b], gsems[b]).wait()
            pltpu.make_async_copy(
                pe_hbm.at[pl.ds(pbase, CHUNK)], pebs[b], psems[b]).wait()

            rbuf = rows[b]
            pbuf = pebs[b]

            def fold(r, _):
                for k in range(VPR):
                    v = rbuf[r, pl.ds(k * LANES, LANES)]
                    plsc.addupdate(pbuf.at[r, pl.ds(k * LANES, LANES)],
                                   v * _SCALE)
                return 0

            lax.fori_loop(0, CHUNK, fold, 0)

            pltpu.async_copy(pbuf, out_hbm.at[pl.ds(tbase + j * CHUNK, CHUNK)],
                             osems[b])
        return 0

    lax.fori_loop(0, NCHUNK // NBUF, group_step, 0)

    for b in range(NBUF):
        pltpu.make_async_copy(pebs[b], out_hbm.at[pl.ds(tbase, CHUNK)],
                              osems[b]).wait()


@jax.jit
def _emb_lookup(emb, idx, pe):
    mesh = plsc.VectorSubcoreMesh(core_axis_name="c", subcore_axis_name="s")
    kfn = pl.kernel(
        _emb_body,
        mesh=mesh,
        out_type=jax.ShapeDtypeStruct((NTOK, D_MODEL), jnp.float32),
        scratch_types=(
            [pltpu.VMEM((TPW,), jnp.int32)]
            + [pltpu.VMEM((CHUNK, D_MODEL), jnp.float32)] * (2 * NBUF)
            + [pltpu.SemaphoreType.DMA] * (3 * NBUF)
        ),
    )
    return kfn(emb, idx, pe)


def kernel(x, emb):
    idx = x.reshape(NTOK).astype(jnp.int32)
    pe = jnp.asarray(_PE)
    out = _emb_lookup(emb, idx, pe)
    return out.reshape(BATCH, SEQ, D_MODEL)
